# baseline (speedup 1.0000x reference)
import functools

import jax
import jax.numpy as jnp
import numpy as np
from jax.experimental import pallas as pl
from jax.experimental.pallas import tpu as pltpu

N = 10000
B = 1792
NB = -(-N // B)
KW = N - (NB - 1) * B
EPS = 1e-5


def _build_schedule():
    rows = []
    for c in range(NB):
        js = [j for j in range(NB) if j != c] + [c]
        for pos, j in enumerate(js):
            last = 1 if pos == NB - 1 else 0
            do_l2 = 1 if (j < c or last) else 0
            fin_e = 1 if (last and c == NB - 1) else 0
            rows.append((0, c, j, c if fin_e else NB - 1, last, do_l2,
                         fin_e))
        for k in range(c - 1, -1, -1):
            fin = 1 if c == NB - 1 else 0
            rows.append((1, k, c, k if fin else NB - 1, 0, 1, fin))
    return np.asarray(rows, dtype=np.int32)


_SCHED = _build_schedule()
_NSTEPS = _SCHED.shape[0]


def _supports_body(x_ref, xbi_ref, w1_ref, out_ref):
    w1 = w1_ref[...]
    a = jnp.dot(x_ref[...], w1, preferred_element_type=jnp.float32)
    b = jnp.dot(xbi_ref[...], w1, preferred_element_type=jnp.float32)
    out_ref[...] = jnp.concatenate([a, b], axis=1)


def _fused_body(idx_ref, sadj_ref, s1_ref, b1c_ref, w2c_ref,
                e_ref, eacc_ref, h2_ref, hpre_ref):
    t = pl.program_id(0)
    phase = idx_ref[t, 0]
    i = idx_ref[t, 1]
    j = idx_ref[t, 2]
    last_in_row = idx_ref[t, 4]
    do_l2 = idx_ref[t, 5]
    fin_e = idx_ref[t, 6]
    blk = sadj_ref[...]

    @pl.when(t == 0)
    def _zero_scratch():
        eacc_ref[...] = jnp.zeros_like(eacc_ref)
        hpre_ref[...] = jnp.zeros_like(hpre_ref)

    @pl.when((phase == 0) & (j < NB - 1))
    def _layer1_full():
        s1j = s1_ref[pl.ds(j * B, B), :]
        hpre_ref[...] += jnp.dot(blk, s1j,
                                 preferred_element_type=jnp.float32)

    @pl.when((phase == 0) & (j == NB - 1))
    def _layer1_ragged():
        s1j = s1_ref[pl.ds(j * B, KW), :]
        hpre_ref[...] += jnp.dot(blk[:, :KW], s1j,
                                 preferred_element_type=jnp.float32)

    @pl.when(last_in_row == 1)
    def _finalize_h2():
        h = jax.nn.relu(hpre_ref[...] + b1c_ref[...])
        h2 = jnp.dot(h, w2c_ref[...], preferred_element_type=jnp.float32)
        h2_ref[pl.ds(i * B, B), :] = h2
        hpre_ref[...] = jnp.zeros_like(hpre_ref)

    @pl.when((do_l2 == 1) & (j < NB - 1))
    def _layer2_full():
        h2j = h2_ref[pl.ds(j * B, B), :]
        eacc_ref[pl.ds(i * B, B), :] += jnp.dot(
            blk, h2j, preferred_element_type=jnp.float32)

    @pl.when((do_l2 == 1) & (j == NB - 1))
    def _layer2_ragged():
        h2j = h2_ref[pl.ds(j * B, KW), :]
        eacc_ref[pl.ds(i * B, B), :] += jnp.dot(
            blk[:, :KW], h2j, preferred_element_type=jnp.float32)

    @pl.when(fin_e == 1)
    def _finalize_row():
        e_ref[...] = eacc_ref[pl.ds(i * B, B), :]


def _post_body(e_ref, b2c_ref, dw1p_ref, db1_ref, dw2_ref, db2_ref,
               zwp_ref, zb_ref, g_ref, bta_ref, piw_ref, pib_ref,
               dw_ref, db_ref, mw_ref, mb_ref,
               emb_ref, embbi_ref, de_ref, pi_ref, disp_ref, mean_ref):
    e = e_ref[...] + b2c_ref[...]
    emb_ref[...] = e[:, :32]
    embbi_ref[...] = e[:, 32:]
    d1 = jax.nn.relu(
        jnp.dot(e, dw1p_ref[...], preferred_element_type=jnp.float32)
        + db1_ref[...])
    de_ref[...] = (jnp.dot(d1, dw2_ref[...],
                           preferred_element_type=jnp.float32)
                   + db2_ref[...])
    z = (jnp.dot(e, zwp_ref[...], preferred_element_type=jnp.float32)
         + zb_ref[...])
    mu = jnp.mean(z, axis=0, keepdims=True)
    var = jnp.mean((z - mu) ** 2, axis=0, keepdims=True)
    zn = (z - mu) / jnp.sqrt(var + EPS) * g_ref[...] + bta_ref[...]
    zr = jax.nn.relu(zn)
    pi_ref[...] = jax.nn.sigmoid(
        jnp.dot(zr, piw_ref[...], preferred_element_type=jnp.float32)
        + pib_ref[...])
    t = (jnp.dot(zr, dw_ref[...], preferred_element_type=jnp.float32)
         + db_ref[...])
    sp = jnp.maximum(t, 0.0) + jnp.log1p(jnp.exp(-jnp.abs(t)))
    disp_ref[...] = jnp.clip(sp, 0.0001, 10000.0)
    m = (jnp.dot(zr, mw_ref[...], preferred_element_type=jnp.float32)
         + mb_ref[...])
    mean_ref[...] = jnp.clip(jnp.exp(m), 1e-05, 1000000.0)


@jax.jit
def kernel(x, x_bi, sadj, W1, b1, W2, b2, dec_W1, dec_b1, dec_W2, dec_b2,
           zW, zb, bn_gamma, bn_beta, piW, pib, dispW, dispb, meanW, meanb):
    f32 = jnp.float32

    b1c = jnp.concatenate([b1, b1]).reshape(1, 128)
    w2c = jnp.zeros((128, 64), f32).at[:64, :32].set(W2).at[64:, 32:].set(W2)
    b2c = jnp.concatenate([b2, b2]).reshape(1, 64)
    dw1p = jnp.zeros((64, 64), f32).at[:32, :].set(dec_W1)
    zwp = jnp.zeros((64, 64), f32).at[32:, :].set(zW)
    sched = jnp.asarray(_SCHED)

    s1cat = pl.pallas_call(
        _supports_body,
        out_shape=jax.ShapeDtypeStruct((N, 128), f32),
    )(x, x_bi, W1)

    cst = lambda t, idx: (0, 0)
    e = pl.pallas_call(
        _fused_body,
        grid_spec=pltpu.PrefetchScalarGridSpec(
            num_scalar_prefetch=1,
            grid=(_NSTEPS,),
            in_specs=[
                pl.BlockSpec((B, B), lambda t, idx: (idx[t, 1], idx[t, 2])),
                pl.BlockSpec((N, 128), cst),
                pl.BlockSpec((1, 128), cst),
                pl.BlockSpec((128, 64), cst),
            ],
            out_specs=pl.BlockSpec((B, 64), lambda t, idx: (idx[t, 3], 0)),
            scratch_shapes=[
                pltpu.VMEM((NB * B, 64), jnp.float32),
                pltpu.VMEM((NB * B, 64), jnp.float32),
                pltpu.VMEM((B, 128), jnp.float32),
            ],
        ),
        out_shape=jax.ShapeDtypeStruct((N, 64), f32),
        compiler_params=pltpu.CompilerParams(
            dimension_semantics=("arbitrary",)),
    )(sched, sadj, s1cat, b1c, w2c)

    emb, emb_bi, de_emb, pi, disp, mean = pl.pallas_call(
        _post_body,
        out_shape=[
            jax.ShapeDtypeStruct((N, 32), f32),
            jax.ShapeDtypeStruct((N, 32), f32),
            jax.ShapeDtypeStruct((N, 128), f32),
            jax.ShapeDtypeStruct((N, 128), f32),
            jax.ShapeDtypeStruct((N, 128), f32),
            jax.ShapeDtypeStruct((N, 128), f32),
        ],
    )(e, b2c, dw1p, dec_b1.reshape(1, 64), dec_W2, dec_b2.reshape(1, 128),
      zwp, zb.reshape(1, 64), bn_gamma.reshape(1, 64), bn_beta.reshape(1, 64),
      piW, pib.reshape(1, 128), dispW, dispb.reshape(1, 128), meanW,
      meanb.reshape(1, 128))

    return (emb, emb_bi, de_emb, pi, disp, mean)


# speedup vs baseline: 1.0266x; 1.0266x over previous
import functools

import jax
import jax.numpy as jnp
import numpy as np
from jax.experimental import pallas as pl
from jax.experimental.pallas import tpu as pltpu

N = 10000
B = 1792
NB = -(-N // B)
KW = N - (NB - 1) * B
EPS = 1e-5


def _build_schedule():
    rows = []
    for c in range(NB):
        js = [j for j in range(NB) if j != c] + [c]
        for pos, j in enumerate(js):
            last = 1 if pos == NB - 1 else 0
            do_l2 = 1 if (j < c or last) else 0
            fin_e = 1 if (last and c == NB - 1) else 0
            rows.append((0, c, j, c if fin_e else NB - 1, last, do_l2,
                         fin_e))
        for k in range(c - 1, -1, -1):
            fin = 1 if c == NB - 1 else 0
            rows.append((1, k, c, k if fin else NB - 1, 0, 1, fin))
    return np.asarray(rows, dtype=np.int32)


_SCHED = _build_schedule()
_NSTEPS = _SCHED.shape[0]


def _supports_body(x_ref, xbi_ref, w1_ref, out_ref):
    w1 = w1_ref[...]
    a = jnp.dot(x_ref[...], w1, preferred_element_type=jnp.float32)
    b = jnp.dot(xbi_ref[...], w1, preferred_element_type=jnp.float32)
    out_ref[...] = jnp.concatenate([a, b], axis=1).astype(jnp.bfloat16)


def _fused_body(idx_ref, sadj_ref, s1_ref, b1c_ref, w2c_ref,
                e_ref, eacc_ref, h2_ref, hpre_ref):
    t = pl.program_id(0)
    phase = idx_ref[t, 0]
    i = idx_ref[t, 1]
    j = idx_ref[t, 2]
    last_in_row = idx_ref[t, 4]
    do_l2 = idx_ref[t, 5]
    fin_e = idx_ref[t, 6]
    blk = sadj_ref[...].astype(jnp.bfloat16)

    @pl.when(t == 0)
    def _zero_scratch():
        eacc_ref[...] = jnp.zeros_like(eacc_ref)
        hpre_ref[...] = jnp.zeros_like(hpre_ref)

    @pl.when((phase == 0) & (j < NB - 1))
    def _layer1_full():
        s1j = s1_ref[pl.ds(j * B, B), :]
        hpre_ref[...] += jnp.dot(blk, s1j,
                                 preferred_element_type=jnp.float32)

    @pl.when((phase == 0) & (j == NB - 1))
    def _layer1_ragged():
        s1j = s1_ref[pl.ds(j * B, KW), :]
        hpre_ref[...] += jnp.dot(blk[:, :KW], s1j,
                                 preferred_element_type=jnp.float32)

    @pl.when(last_in_row == 1)
    def _finalize_h2():
        h = jax.nn.relu(hpre_ref[...] + b1c_ref[...])
        h2 = jnp.dot(h, w2c_ref[...], preferred_element_type=jnp.float32)
        h2_ref[pl.ds(i * B, B), :] = h2.astype(jnp.bfloat16)
        hpre_ref[...] = jnp.zeros_like(hpre_ref)

    @pl.when((do_l2 == 1) & (j < NB - 1))
    def _layer2_full():
        h2j = h2_ref[pl.ds(j * B, B), :]
        eacc_ref[pl.ds(i * B, B), :] += jnp.dot(
            blk, h2j, preferred_element_type=jnp.float32)

    @pl.when((do_l2 == 1) & (j == NB - 1))
    def _layer2_ragged():
        h2j = h2_ref[pl.ds(j * B, KW), :]
        eacc_ref[pl.ds(i * B, B), :] += jnp.dot(
            blk[:, :KW], h2j, preferred_element_type=jnp.float32)

    @pl.when(fin_e == 1)
    def _finalize_row():
        e_ref[...] = eacc_ref[pl.ds(i * B, B), :]


def _post_body(e_ref, b2c_ref, dw1p_ref, db1_ref, dw2_ref, db2_ref,
               zwp_ref, zb_ref, g_ref, bta_ref, piw_ref, pib_ref,
               dw_ref, db_ref, mw_ref, mb_ref,
               emb_ref, embbi_ref, de_ref, pi_ref, disp_ref, mean_ref):
    e = e_ref[...] + b2c_ref[...]
    emb_ref[...] = e[:, :32]
    embbi_ref[...] = e[:, 32:]
    d1 = jax.nn.relu(
        jnp.dot(e, dw1p_ref[...], preferred_element_type=jnp.float32)
        + db1_ref[...])
    de_ref[...] = (jnp.dot(d1, dw2_ref[...],
                           preferred_element_type=jnp.float32)
                   + db2_ref[...])
    z = (jnp.dot(e, zwp_ref[...], preferred_element_type=jnp.float32)
         + zb_ref[...])
    mu = jnp.mean(z, axis=0, keepdims=True)
    var = jnp.mean((z - mu) ** 2, axis=0, keepdims=True)
    zn = (z - mu) / jnp.sqrt(var + EPS) * g_ref[...] + bta_ref[...]
    zr = jax.nn.relu(zn)
    pi_ref[...] = jax.nn.sigmoid(
        jnp.dot(zr, piw_ref[...], preferred_element_type=jnp.float32)
        + pib_ref[...])
    t = (jnp.dot(zr, dw_ref[...], preferred_element_type=jnp.float32)
         + db_ref[...])
    sp = jnp.maximum(t, 0.0) + jnp.log1p(jnp.exp(-jnp.abs(t)))
    disp_ref[...] = jnp.clip(sp, 0.0001, 10000.0)
    m = (jnp.dot(zr, mw_ref[...], preferred_element_type=jnp.float32)
         + mb_ref[...])
    mean_ref[...] = jnp.clip(jnp.exp(m), 1e-05, 1000000.0)


@jax.jit
def kernel(x, x_bi, sadj, W1, b1, W2, b2, dec_W1, dec_b1, dec_W2, dec_b2,
           zW, zb, bn_gamma, bn_beta, piW, pib, dispW, dispb, meanW, meanb):
    f32 = jnp.float32

    b1c = jnp.concatenate([b1, b1]).reshape(1, 128)
    w2c = jnp.zeros((128, 64), f32).at[:64, :32].set(W2).at[64:, 32:].set(W2)
    b2c = jnp.concatenate([b2, b2]).reshape(1, 64)
    dw1p = jnp.zeros((64, 64), f32).at[:32, :].set(dec_W1)
    zwp = jnp.zeros((64, 64), f32).at[32:, :].set(zW)
    sched = jnp.asarray(_SCHED)

    s1cat = pl.pallas_call(
        _supports_body,
        out_shape=jax.ShapeDtypeStruct((N, 128), jnp.bfloat16),
    )(x, x_bi, W1)

    cst = lambda t, idx: (0, 0)
    e = pl.pallas_call(
        _fused_body,
        grid_spec=pltpu.PrefetchScalarGridSpec(
            num_scalar_prefetch=1,
            grid=(_NSTEPS,),
            in_specs=[
                pl.BlockSpec((B, B), lambda t, idx: (idx[t, 1], idx[t, 2])),
                pl.BlockSpec((N, 128), cst),
                pl.BlockSpec((1, 128), cst),
                pl.BlockSpec((128, 64), cst),
            ],
            out_specs=pl.BlockSpec((B, 64), lambda t, idx: (idx[t, 3], 0)),
            scratch_shapes=[
                pltpu.VMEM((NB * B, 64), jnp.float32),
                pltpu.VMEM((NB * B, 64), jnp.bfloat16),
                pltpu.VMEM((B, 128), jnp.float32),
            ],
        ),
        out_shape=jax.ShapeDtypeStruct((N, 64), f32),
        compiler_params=pltpu.CompilerParams(
            dimension_semantics=("arbitrary",)),
    )(sched, sadj, s1cat, b1c, w2c)

    emb, emb_bi, de_emb, pi, disp, mean = pl.pallas_call(
        _post_body,
        out_shape=[
            jax.ShapeDtypeStruct((N, 32), f32),
            jax.ShapeDtypeStruct((N, 32), f32),
            jax.ShapeDtypeStruct((N, 128), f32),
            jax.ShapeDtypeStruct((N, 128), f32),
            jax.ShapeDtypeStruct((N, 128), f32),
            jax.ShapeDtypeStruct((N, 128), f32),
        ],
    )(e, b2c, dw1p, dec_b1.reshape(1, 64), dec_W2, dec_b2.reshape(1, 128),
      zwp, zb.reshape(1, 64), bn_gamma.reshape(1, 64), bn_beta.reshape(1, 64),
      piW, pib.reshape(1, 128), dispW, dispb.reshape(1, 128), meanW,
      meanb.reshape(1, 128))

    return (emb, emb_bi, de_emb, pi, disp, mean)


# speedup vs baseline: 1.1508x; 1.1211x over previous
import functools

import jax
import jax.numpy as jnp
import numpy as np
from jax.experimental import pallas as pl
from jax.experimental.pallas import tpu as pltpu

N = 10000
B = 2048
NB = -(-N // B)
KW = N - (NB - 1) * B
EPS = 1e-5


def _build_schedule():
    rows = []
    for c in range(NB):
        js = [j for j in range(NB) if j != c] + [c]
        for pos, j in enumerate(js):
            last = 1 if pos == NB - 1 else 0
            do_l2 = 1 if (j < c or last) else 0
            fin_e = 1 if (last and c == NB - 1) else 0
            rows.append((0, c, j, c if fin_e else NB - 1, last, do_l2,
                         fin_e))
        for k in range(c - 1, -1, -1):
            fin = 1 if c == NB - 1 else 0
            rows.append((1, k, c, k if fin else NB - 1, 0, 1, fin))
    return np.asarray(rows, dtype=np.int32)


_SCHED = _build_schedule()
_NSTEPS = _SCHED.shape[0]


def _supports_body(x_ref, xbi_ref, w1_ref, out_ref):
    w1 = w1_ref[...]
    a = jnp.dot(x_ref[...], w1, preferred_element_type=jnp.float32)
    b = jnp.dot(xbi_ref[...], w1, preferred_element_type=jnp.float32)
    out_ref[...] = jnp.concatenate([a, b], axis=1).astype(jnp.bfloat16)


def _fused_body(idx_ref, sadj_ref, s1_ref, b1r_ref, w2_ref,
                e_ref, eacc_ref, h2_ref, hpre_ref):
    t = pl.program_id(0)
    phase = idx_ref[t, 0]
    i = idx_ref[t, 1]
    j = idx_ref[t, 2]
    last_in_row = idx_ref[t, 4]
    do_l2 = idx_ref[t, 5]
    fin_e = idx_ref[t, 6]
    blk = sadj_ref[...].astype(jnp.bfloat16)

    @pl.when(t == 0)
    def _zero_scratch():
        eacc_ref[...] = jnp.zeros_like(eacc_ref)
        hpre_ref[...] = jnp.zeros_like(hpre_ref)

    @pl.when((phase == 0) & (j < NB - 1))
    def _layer1_full():
        s1j = s1_ref[pl.ds(j * B, B), :]
        hpre_ref[...] += jnp.dot(blk, s1j,
                                 preferred_element_type=jnp.float32)

    @pl.when((phase == 0) & (j == NB - 1))
    def _layer1_ragged():
        s1j = s1_ref[pl.ds(j * B, KW), :]
        hpre_ref[...] += jnp.dot(blk[:, :KW], s1j,
                                 preferred_element_type=jnp.float32)

    @pl.when(last_in_row == 1)
    def _finalize_h2():
        b1r = b1r_ref[...]
        w2 = w2_ref[...]
        h = jax.nn.relu(hpre_ref[...]
                        + jnp.concatenate([b1r, b1r], axis=1))
        h2 = jnp.concatenate(
            [jnp.dot(h[:, :64], w2, preferred_element_type=jnp.float32),
             jnp.dot(h[:, 64:], w2, preferred_element_type=jnp.float32)],
            axis=1)
        h2_ref[pl.ds(i * B, B), :] = h2.astype(jnp.bfloat16)
        hpre_ref[...] = jnp.zeros_like(hpre_ref)

    @pl.when((do_l2 == 1) & (j < NB - 1))
    def _layer2_full():
        h2j = h2_ref[pl.ds(j * B, B), :]
        eacc_ref[pl.ds(i * B, B), :] += jnp.dot(
            blk, h2j, preferred_element_type=jnp.float32)

    @pl.when((do_l2 == 1) & (j == NB - 1))
    def _layer2_ragged():
        h2j = h2_ref[pl.ds(j * B, KW), :]
        eacc_ref[pl.ds(i * B, B), :] += jnp.dot(
            blk[:, :KW], h2j, preferred_element_type=jnp.float32)

    @pl.when(fin_e == 1)
    def _finalize_row():
        e_ref[...] = eacc_ref[pl.ds(i * B, B), :]


def _post_body(e_ref, b2r_ref, dw1_ref, db1_ref, dw2_ref, db2_ref,
               zw_ref, zb_ref, g_ref, bta_ref, piw_ref, pib_ref,
               dw_ref, db_ref, mw_ref, mb_ref,
               emb_ref, embbi_ref, de_ref, pi_ref, disp_ref, mean_ref):
    e = e_ref[...]
    b2r = b2r_ref[...]
    esp = e[:, :32] + b2r
    ebi = e[:, 32:] + b2r
    emb_ref[...] = esp
    embbi_ref[...] = ebi
    d1 = jax.nn.relu(
        jnp.dot(esp, dw1_ref[...], preferred_element_type=jnp.float32)
        + db1_ref[...])
    de_ref[...] = (jnp.dot(d1, dw2_ref[...],
                           preferred_element_type=jnp.float32)
                   + db2_ref[...])
    z = (jnp.dot(ebi, zw_ref[...], preferred_element_type=jnp.float32)
         + zb_ref[...])
    mu = jnp.mean(z, axis=0, keepdims=True)
    var = jnp.mean((z - mu) ** 2, axis=0, keepdims=True)
    zn = (z - mu) / jnp.sqrt(var + EPS) * g_ref[...] + bta_ref[...]
    zr = jax.nn.relu(zn)
    pi_ref[...] = jax.nn.sigmoid(
        jnp.dot(zr, piw_ref[...], preferred_element_type=jnp.float32)
        + pib_ref[...])
    t = (jnp.dot(zr, dw_ref[...], preferred_element_type=jnp.float32)
         + db_ref[...])
    sp = jnp.maximum(t, 0.0) + jnp.log1p(jnp.exp(-jnp.abs(t)))
    disp_ref[...] = jnp.clip(sp, 0.0001, 10000.0)
    m = (jnp.dot(zr, mw_ref[...], preferred_element_type=jnp.float32)
         + mb_ref[...])
    mean_ref[...] = jnp.clip(jnp.exp(m), 1e-05, 1000000.0)


@jax.jit
def kernel(x, x_bi, sadj, W1, b1, W2, b2, dec_W1, dec_b1, dec_W2, dec_b2,
           zW, zb, bn_gamma, bn_beta, piW, pib, dispW, dispb, meanW, meanb):
    f32 = jnp.float32

    sched = jnp.asarray(_SCHED)

    s1cat = pl.pallas_call(
        _supports_body,
        out_shape=jax.ShapeDtypeStruct((N, 128), jnp.bfloat16),
    )(x, x_bi, W1)

    cst = lambda t, idx: (0, 0)
    e = pl.pallas_call(
        _fused_body,
        grid_spec=pltpu.PrefetchScalarGridSpec(
            num_scalar_prefetch=1,
            grid=(_NSTEPS,),
            in_specs=[
                pl.BlockSpec((B, B), lambda t, idx: (idx[t, 1], idx[t, 2])),
                pl.BlockSpec((N, 128), cst),
                pl.BlockSpec((1, 64), cst),
                pl.BlockSpec((64, 32), cst),
            ],
            out_specs=pl.BlockSpec((B, 64), lambda t, idx: (idx[t, 3], 0)),
            scratch_shapes=[
                pltpu.VMEM((NB * B, 64), jnp.float32),
                pltpu.VMEM((NB * B, 64), jnp.bfloat16),
                pltpu.VMEM((B, 128), jnp.float32),
            ],
        ),
        out_shape=jax.ShapeDtypeStruct((N, 64), f32),
        compiler_params=pltpu.CompilerParams(
            dimension_semantics=("arbitrary",)),
    )(sched, sadj, s1cat, b1.reshape(1, 64), W2)

    emb, emb_bi, de_emb, pi, disp, mean = pl.pallas_call(
        _post_body,
        out_shape=[
            jax.ShapeDtypeStruct((N, 32), f32),
            jax.ShapeDtypeStruct((N, 32), f32),
            jax.ShapeDtypeStruct((N, 128), f32),
            jax.ShapeDtypeStruct((N, 128), f32),
            jax.ShapeDtypeStruct((N, 128), f32),
            jax.ShapeDtypeStruct((N, 128), f32),
        ],
    )(e, b2.reshape(1, 32), dec_W1, dec_b1.reshape(1, 64), dec_W2,
      dec_b2.reshape(1, 128), zW, zb.reshape(1, 64),
      bn_gamma.reshape(1, 64), bn_beta.reshape(1, 64),
      piW, pib.reshape(1, 128), dispW, dispb.reshape(1, 128), meanW,
      meanb.reshape(1, 128))

    return (emb, emb_bi, de_emb, pi, disp, mean)
